# baseline (speedup 1.0000x reference)
import functools

import jax
import jax.numpy as jnp
from jax import lax
from jax.experimental import pallas as pl
from jax.experimental.pallas import tpu as pltpu
from jax.experimental.pallas import tpu_sc as plsc

B, D, H, N = 1024, 64, 512, 100000
TN = 4000
NSTEPS = N // TN
NC = TN // 8
BIG_I32 = 2**30
BIG_F32 = 3.0e38


def _argmin_body(table_hbm, codesneg_ref, q2_ref, t2_ref, out_ref,
                 best_d, best_i, tbuf, sems):
    i = pl.program_id(0)

    def tile_cp(j, slot):
        return pltpu.make_async_copy(
            table_hbm.at[pl.ds(j * TN, TN), :], tbuf.at[slot], sems.at[slot])

    @pl.when(i == 0)
    def _prime():
        tile_cp(0, 0).start()

    @pl.when(i + 1 < NSTEPS)
    def _prefetch():
        tile_cp(i + 1, (i + 1) % 2).start()

    tile_cp(i, i % 2).wait()
    dT = lax.dot_general(
        tbuf[i % 2], codesneg_ref[...],
        (((1,), (1,)), ((), ())),
        preferred_element_type=jnp.float32,
    )
    d2 = (q2_ref[...] + dT) + t2_ref[...]
    d2r = d2.reshape(NC, 8, B)
    m8 = jnp.min(d2r, axis=0)
    eq = d2r == m8[None, :, :]
    cid = lax.broadcasted_iota(jnp.int32, (NC, 8, B), 0)
    cmin = jnp.min(jnp.where(eq, cid, BIG_I32), axis=0)
    sub = lax.broadcasted_iota(jnp.int32, (8, B), 0)
    rows8 = cmin * 8 + sub + i * TN
    m = jnp.min(m8, axis=0, keepdims=True)
    cand = jnp.where(m8 == m, rows8, BIG_I32)
    idx = jnp.min(cand, axis=0, keepdims=True)

    @pl.when(i == 0)
    def _init():
        best_d[...] = m
        best_i[...] = idx

    @pl.when(i > 0)
    def _update():
        better = m < best_d[...]
        best_d[...] = jnp.where(better, m, best_d[...])
        best_i[...] = jnp.where(better, idx, best_i[...])

    @pl.when(i == NSTEPS - 1)
    def _emit():
        out_ref[...] = best_i[...]


def _argmin_tc(codesneg, codes_table, q2_row, t2_col):
    return pl.pallas_call(
        _argmin_body,
        grid=(NSTEPS,),
        in_specs=[
            pl.BlockSpec(memory_space=pl.ANY),
            pl.BlockSpec((B, D), lambda i: (0, 0)),
            pl.BlockSpec((1, B), lambda i: (0, 0)),
            pl.BlockSpec((TN, 1), lambda i: (i, 0)),
        ],
        out_specs=pl.BlockSpec((1, B), lambda i: (0, 0)),
        out_shape=jax.ShapeDtypeStruct((1, B), jnp.int32),
        scratch_shapes=[
            pltpu.VMEM((1, B), jnp.float32),
            pltpu.VMEM((1, B), jnp.int32),
            pltpu.VMEM((2, TN, D), jnp.float32),
            pltpu.SemaphoreType.DMA((2,)),
        ],
    )(codes_table, codesneg, q2_row, t2_col)


def _gather_sc(table128, chosen):
    info = plsc.get_sparse_core_info()
    nw = info.num_cores * info.num_subcores
    b_per_w = B // nw
    mesh = plsc.VectorSubcoreMesh(core_axis_name="c", subcore_axis_name="s")

    @functools.partial(
        pl.kernel,
        out_type=jax.ShapeDtypeStruct((B, 2 * D), jnp.float32),
        mesh=mesh,
        scratch_types=[
            pltpu.VMEM((b_per_w,), jnp.int32),
            pltpu.VMEM((b_per_w,), jnp.int32),
            pltpu.VMEM((b_per_w, 2 * D), jnp.float32),
            pltpu.SemaphoreType.DMA,
        ],
    )
    def k(table_hbm, idx_hbm, out_hbm, idx_v, pidx_v, rows_v, sem):
        wid = lax.axis_index("s") * info.num_cores + lax.axis_index("c")
        base = wid * b_per_w
        pltpu.sync_copy(idx_hbm.at[pl.ds(base, b_per_w)], idx_v)
        for j in range(b_per_w // 16):
            sl = pl.ds(j * 16, 16)
            pidx_v[sl] = lax.shift_right_logical(idx_v[sl], 1)
        pltpu.async_copy(table_hbm.at[pidx_v], rows_v, sem).wait()
        pltpu.sync_copy(rows_v, out_hbm.at[pl.ds(base, b_per_w)])

    return k(table128, chosen)


def _mlp_body(packed_ref, chosen_ref, w1_ref, b1_ref, wu_ref, bu_ref, ws_ref,
              bs_ref, mu_ref, std_ref):
    odd = (chosen_ref[...] & 1) == 1
    prev = jnp.where(odd, packed_ref[:, D:], packed_ref[:, :D])
    h1 = lax.dot_general(
        prev, w1_ref[...], (((1,), (1,)), ((), ())),
        preferred_element_type=jnp.float32,
    ) + b1_ref[...]
    h1 = jnp.maximum(h1, 0.0)
    mu_ref[...] = lax.dot_general(
        h1, wu_ref[...], (((1,), (1,)), ((), ())),
        preferred_element_type=jnp.float32,
    ) + bu_ref[...]
    s = lax.dot_general(
        h1, ws_ref[...], (((1,), (1,)), ((), ())),
        preferred_element_type=jnp.float32,
    ) + bs_ref[...]
    std_ref[...] = jax.nn.softplus(s) + 0.0001


def _mlp_tc(packed, chosen2d, W1, b1r, Wu, bur, Ws, bsr):
    return pl.pallas_call(
        _mlp_body,
        out_shape=(
            jax.ShapeDtypeStruct((B, D), jnp.float32),
            jax.ShapeDtypeStruct((B, D), jnp.float32),
        ),
    )(packed, chosen2d, W1, b1r, Wu, bur, Ws, bsr)


def kernel(codes, codes_table, W1, b1, Wu, bu, Ws, bs):
    q2 = jnp.sum(codes * codes, axis=1, keepdims=True)
    t2 = jnp.sum(codes_table * codes_table, axis=1)
    chosen = _argmin_tc(codes * (-2.0), codes_table,
                        q2.reshape(1, B), t2.reshape(N, 1))
    packed = _gather_sc(codes_table.reshape(N // 2, 2 * D), chosen.reshape(B))
    mu, std = _mlp_tc(packed, chosen.reshape(B, 1), W1, b1.reshape(1, H),
                      Wu, bu.reshape(1, D), Ws, bs.reshape(1, D))
    return (mu, std)


# speedup vs baseline: 1.2307x; 1.2307x over previous
import functools

import jax
import jax.numpy as jnp
from jax import lax
from jax.experimental import pallas as pl
from jax.experimental.pallas import tpu as pltpu
from jax.experimental.pallas import tpu_sc as plsc

B, D, H, N = 1024, 64, 512, 100000
TN = 4000
NSTEPS = N // TN
NC = TN // 8
BIG_I32 = 2**30
BIG_F32 = 3.0e38


def _argmin_body(table_hbm, codesneg_ref, q2_ref, t2_ref, iota_ref, out_ref,
                 best_d, best_i, tbuf, sems):
    i = pl.program_id(0)

    def tile_cp(j, slot):
        return pltpu.make_async_copy(
            table_hbm.at[pl.ds(j * TN, TN), :], tbuf.at[slot], sems.at[slot])

    @pl.when(i == 0)
    def _prime():
        tile_cp(0, 0).start()

    @pl.when(i + 1 < NSTEPS)
    def _prefetch():
        tile_cp(i + 1, (i + 1) % 2).start()

    tile_cp(i, i % 2).wait()
    dT = lax.dot_general(
        codesneg_ref[...], tbuf[i % 2],
        (((1,), (1,)), ((), ())),
        preferred_element_type=jnp.float32,
    )
    d2 = (q2_ref[...] + dT) + t2_ref[0]
    m = jnp.min(d2, axis=1, keepdims=True)
    cand = jnp.where(d2 == m, iota_ref[0], BIG_F32)
    idx = jnp.min(cand, axis=1, keepdims=True)

    @pl.when(i == 0)
    def _init():
        best_d[...] = m
        best_i[...] = idx

    @pl.when(i > 0)
    def _update():
        better = m < best_d[...]
        best_d[...] = jnp.where(better, m, best_d[...])
        best_i[...] = jnp.where(better, idx, best_i[...])

    @pl.when(i == NSTEPS - 1)
    def _emit():
        out_ref[...] = best_i[...].astype(jnp.int32)


def _argmin_tc(codesneg, codes_table, q2_col, t2_row, iota_row):
    return pl.pallas_call(
        _argmin_body,
        grid=(NSTEPS,),
        in_specs=[
            pl.BlockSpec(memory_space=pl.ANY),
            pl.BlockSpec((B, D), lambda i: (0, 0)),
            pl.BlockSpec((B, 1), lambda i: (0, 0)),
            pl.BlockSpec((1, 1, TN), lambda i: (i, 0, 0)),
            pl.BlockSpec((1, 1, TN), lambda i: (i, 0, 0)),
        ],
        out_specs=pl.BlockSpec((B, 1), lambda i: (0, 0)),
        out_shape=jax.ShapeDtypeStruct((B, 1), jnp.int32),
        scratch_shapes=[
            pltpu.VMEM((B, 1), jnp.float32),
            pltpu.VMEM((B, 1), jnp.float32),
            pltpu.VMEM((2, TN, D), jnp.float32),
            pltpu.SemaphoreType.DMA((2,)),
        ],
    )(codes_table, codesneg, q2_col, t2_row, iota_row)


def _gather_sc(table128, chosen):
    info = plsc.get_sparse_core_info()
    nw = info.num_cores * info.num_subcores
    b_per_w = B // nw
    mesh = plsc.VectorSubcoreMesh(core_axis_name="c", subcore_axis_name="s")

    @functools.partial(
        pl.kernel,
        out_type=jax.ShapeDtypeStruct((B, 2 * D), jnp.float32),
        mesh=mesh,
        scratch_types=[
            pltpu.VMEM((b_per_w,), jnp.int32),
            pltpu.VMEM((b_per_w,), jnp.int32),
            pltpu.VMEM((b_per_w, 2 * D), jnp.float32),
            pltpu.SemaphoreType.DMA,
        ],
    )
    def k(table_hbm, idx_hbm, out_hbm, idx_v, pidx_v, rows_v, sem):
        wid = lax.axis_index("s") * info.num_cores + lax.axis_index("c")
        base = wid * b_per_w
        pltpu.sync_copy(idx_hbm.at[pl.ds(base, b_per_w)], idx_v)
        for j in range(b_per_w // 16):
            sl = pl.ds(j * 16, 16)
            pidx_v[sl] = lax.shift_right_logical(idx_v[sl], 1)
        pltpu.async_copy(table_hbm.at[pidx_v], rows_v, sem).wait()
        pltpu.sync_copy(rows_v, out_hbm.at[pl.ds(base, b_per_w)])

    return k(table128, chosen)


def _mlp_body(packed_ref, chosen_ref, w1_ref, b1_ref, wu_ref, bu_ref, ws_ref,
              bs_ref, mu_ref, std_ref):
    odd = (chosen_ref[...] & 1) == 1
    prev = jnp.where(odd, packed_ref[:, D:], packed_ref[:, :D])
    h1 = lax.dot_general(
        prev, w1_ref[...], (((1,), (1,)), ((), ())),
        preferred_element_type=jnp.float32,
    ) + b1_ref[...]
    h1 = jnp.maximum(h1, 0.0)
    mu_ref[...] = lax.dot_general(
        h1, wu_ref[...], (((1,), (1,)), ((), ())),
        preferred_element_type=jnp.float32,
    ) + bu_ref[...]
    s = lax.dot_general(
        h1, ws_ref[...], (((1,), (1,)), ((), ())),
        preferred_element_type=jnp.float32,
    ) + bs_ref[...]
    std_ref[...] = jax.nn.softplus(s) + 0.0001


def _mlp_tc(packed, chosen2d, W1, b1r, Wu, bur, Ws, bsr):
    return pl.pallas_call(
        _mlp_body,
        out_shape=(
            jax.ShapeDtypeStruct((B, D), jnp.float32),
            jax.ShapeDtypeStruct((B, D), jnp.float32),
        ),
    )(packed, chosen2d, W1, b1r, Wu, bur, Ws, bsr)


def kernel(codes, codes_table, W1, b1, Wu, bu, Ws, bs):
    q2 = jnp.sum(codes * codes, axis=1, keepdims=True)
    t2 = jnp.sum(codes_table * codes_table, axis=1)
    iota_row = jnp.arange(N, dtype=jnp.float32).reshape(NSTEPS, 1, TN)
    chosen = _argmin_tc(codes * (-2.0), codes_table,
                        q2, t2.reshape(NSTEPS, 1, TN), iota_row)
    packed = _gather_sc(codes_table.reshape(N // 2, 2 * D), chosen.reshape(B))
    mu, std = _mlp_tc(packed, chosen.reshape(B, 1), W1, b1.reshape(1, H),
                      Wu, bu.reshape(1, D), Ws, bs.reshape(1, D))
    return (mu, std)
